# Initial kernel scaffold; baseline (speedup 1.0000x reference)
import jax
import jax.numpy as jnp
from jax.experimental import pallas as pl


def kernel(x, edge_index, W_in, b_in, Ws, bs, eps, W_out, b_out):
    raise NotImplementedError("write your pallas kernel here")


# speedup vs baseline: 5.8048x; 5.8048x over previous
import functools

import jax
import jax.numpy as jnp
from jax import lax
from jax.experimental import pallas as pl
from jax.experimental.pallas import tpu as pltpu
from jax.experimental.pallas import tpu_sc as plsc

N = 10000
E = 320000
D = 128
L = 3
K = 32

NC = 2
NS = 16
NW = NC * NS
EPW = E // NW
CHUNK = 80
NCHUNK = EPW // CHUNK
RPW = 624
ZR = 208

BR = 1000


def _maxk_mask(t):
    cols = lax.broadcasted_iota(jnp.int32, t.shape, 1)
    work = t
    keep = jnp.zeros(t.shape, jnp.bool_)
    neg = jnp.float32(-3.0e38)
    for _ in range(K):
        m = jnp.max(work, axis=1, keepdims=True)
        ismax = work == m
        first = jnp.min(jnp.where(ismax, cols, D), axis=1, keepdims=True)
        sel = cols == first
        keep = jnp.logical_or(keep, sel)
        work = jnp.where(sel, neg, work)
    return jnp.where(keep, t, 0.0)


def _tc_in_body(x_ref, wi_ref, bi_ref, w0_ref, b0_ref, o_ref):
    h0 = jnp.dot(x_ref[...], wi_ref[...], preferred_element_type=jnp.float32)
    h0 = jnp.maximum(h0 + bi_ref[...], 0.0)
    t = jnp.dot(h0, w0_ref[...], preferred_element_type=jnp.float32) + b0_ref[...]
    o_ref[...] = _maxk_mask(t)


def _tc_mid_body(hs_ref, p0_ref, p1_ref, e_ref, w_ref, b_ref, o_ref):
    h = e_ref[...] * hs_ref[...] + p0_ref[...] + p1_ref[...]
    t = jnp.dot(h, w_ref[...], preferred_element_type=jnp.float32) + b_ref[...]
    o_ref[...] = _maxk_mask(t)


def _tc_out_body(hs_ref, p0_ref, p1_ref, e_ref, w_ref, b_ref, o_ref):
    h = e_ref[...] * hs_ref[...] + p0_ref[...] + p1_ref[...]
    o_ref[...] = jnp.dot(h, w_ref[...], preferred_element_type=jnp.float32) + b_ref[...]


def _rows():
    return pl.BlockSpec((BR, D), lambda i: (i, 0))


def _mat():
    return pl.BlockSpec((D, D), lambda i: (0, 0))


def _vec():
    return pl.BlockSpec((1, D), lambda i: (0, 0))


_out_rows = jax.ShapeDtypeStruct((N, D), jnp.float32)

_tc_in = pl.pallas_call(
    _tc_in_body, grid=(N // BR,),
    in_specs=[_rows(), _mat(), _vec(), _mat(), _vec()],
    out_specs=_rows(), out_shape=_out_rows)

_tc_mid = pl.pallas_call(
    _tc_mid_body, grid=(N // BR,),
    in_specs=[_rows(), _rows(), _rows(), _vec(), _mat(), _vec()],
    out_specs=_rows(), out_shape=_out_rows)

_tc_out = pl.pallas_call(
    _tc_out_body, grid=(N // BR,),
    in_specs=[_rows(), _rows(), _rows(), _vec(), _mat(), _vec()],
    out_specs=_rows(), out_shape=_out_rows)


@functools.lru_cache(maxsize=None)
def _make_sc_agg():
    mesh = plsc.VectorSubcoreMesh(core_axis_name="c", subcore_axis_name="s")
    return pl.kernel(
        _sc_agg_body,
        mesh=mesh,
        out_type=jax.ShapeDtypeStruct((NC * N, D), jnp.float32),
        scratch_types=[
            pltpu.VMEM((CHUNK,), jnp.int32),
            pltpu.VMEM((CHUNK,), jnp.int32),
            pltpu.VMEM((CHUNK, D), jnp.float32),
            pltpu.VMEM((ZR, D), jnp.float32),
            pltpu.VMEM_SHARED((N, D), jnp.float32),
            pltpu.SemaphoreType.DMA,
        ],
    )


def _sc_agg_body(hs_hbm, src_hbm, dst_hbm, out_hbm, src_v, dst_v, rows_v, zbuf,
                 acc_sh, sem):
    cid = lax.axis_index("c")
    sid = lax.axis_index("s")
    wg = cid * NS + sid

    def _zrow(r, carry):
        for c in range(D // 16):
            zbuf[r, pl.ds(c * 16, 16)] = jnp.zeros((16,), jnp.float32)
        return carry

    lax.fori_loop(0, ZR, _zrow, 0)

    base = pl.multiple_of(sid * RPW, 16)
    for j in range(RPW // ZR):
        pltpu.sync_copy(zbuf, acc_sh.at[pl.ds(base + j * ZR, ZR)])

    @pl.when(sid == NS - 1)
    def _zero_tail():
        pltpu.sync_copy(zbuf.at[pl.ds(0, 16)],
                        acc_sh.at[pl.ds(NS * RPW, N - NS * RPW)])

    plsc.subcore_barrier()

    def _edges(it, carry):
        e0 = pl.multiple_of(wg * EPW + it * CHUNK, 16)
        pltpu.sync_copy(src_hbm.at[pl.ds(e0, CHUNK)], src_v)
        pltpu.sync_copy(dst_hbm.at[pl.ds(e0, CHUNK)], dst_v)
        pltpu.async_copy(hs_hbm.at[src_v], rows_v, sem).wait()
        pltpu.sync_copy(rows_v, acc_sh.at[dst_v], add=True)
        return carry

    lax.fori_loop(0, NCHUNK, _edges, 0)

    plsc.subcore_barrier()
    obase = pl.multiple_of(cid * N + sid * RPW, 16)
    for j in range(RPW // ZR):
        pltpu.sync_copy(acc_sh.at[pl.ds(base + j * ZR, ZR)],
                        out_hbm.at[pl.ds(obase + j * ZR, ZR)])

    @pl.when(sid == NS - 1)
    def _write_tail():
        pltpu.sync_copy(acc_sh.at[pl.ds(NS * RPW, N - NS * RPW)],
                        out_hbm.at[pl.ds(cid * N + NS * RPW, N - NS * RPW)])


def kernel(x, edge_index, W_in, b_in, Ws, bs, eps, W_out, b_out):
    src = edge_index[0].astype(jnp.int32)
    dst = edge_index[1].astype(jnp.int32)
    ones = jnp.ones((1, D), jnp.float32)

    hs = _tc_in(x, W_in, b_in.reshape(1, D), Ws[0], bs[0].reshape(1, D))
    for i in range(L):
        part = _make_sc_agg()(hs, src, dst)
        p0, p1 = part[:N], part[N:]
        epsb = (1.0 + eps[i]) * ones
        if i < L - 1:
            hs = _tc_mid(hs, p0, p1, epsb, Ws[i + 1], bs[i + 1].reshape(1, D))
        else:
            return _tc_out(hs, p0, p1, epsb, W_out, b_out.reshape(1, D))


# speedup vs baseline: 9.3014x; 1.6024x over previous
import functools

import jax
import jax.numpy as jnp
from jax import lax
from jax.experimental import pallas as pl
from jax.experimental.pallas import tpu as pltpu
from jax.experimental.pallas import tpu_sc as plsc

N = 10000
E = 320000
D = 128
L = 3
K = 32

NC = 2
NS = 16
NW = NC * NS
EPW = E // NW
CHUNK = 80
NCHUNK = EPW // CHUNK
RPW = 624
ZR = 16

BR = 1000


def _maxk_mask(t):
    cols = lax.broadcasted_iota(jnp.int32, t.shape, 1)
    work = t
    keep = jnp.zeros(t.shape, jnp.bool_)
    neg = jnp.float32(-3.0e38)
    for _ in range(K):
        m = jnp.max(work, axis=1, keepdims=True)
        ismax = work == m
        first = jnp.min(jnp.where(ismax, cols, D), axis=1, keepdims=True)
        sel = cols == first
        keep = jnp.logical_or(keep, sel)
        work = jnp.where(sel, neg, work)
    return jnp.where(keep, t, 0.0)


def _tc_in_body(x_ref, wi_ref, bi_ref, w0_ref, b0_ref, o_ref):
    h0 = jnp.dot(x_ref[...], wi_ref[...], preferred_element_type=jnp.float32)
    h0 = jnp.maximum(h0 + bi_ref[...], 0.0)
    t = jnp.dot(h0, w0_ref[...], preferred_element_type=jnp.float32) + b0_ref[...]
    o_ref[...] = _maxk_mask(t)


def _tc_mid_body(hs_ref, p0_ref, p1_ref, e_ref, w_ref, b_ref, o_ref):
    h = e_ref[...] * hs_ref[...] + p0_ref[...] + p1_ref[...]
    t = jnp.dot(h, w_ref[...], preferred_element_type=jnp.float32) + b_ref[...]
    o_ref[...] = _maxk_mask(t)


def _tc_out_body(hs_ref, p0_ref, p1_ref, e_ref, w_ref, b_ref, o_ref):
    h = e_ref[...] * hs_ref[...] + p0_ref[...] + p1_ref[...]
    o_ref[...] = jnp.dot(h, w_ref[...], preferred_element_type=jnp.float32) + b_ref[...]


def _rows():
    return pl.BlockSpec((BR, D), lambda i: (i, 0))


def _mat():
    return pl.BlockSpec((D, D), lambda i: (0, 0))


def _vec():
    return pl.BlockSpec((1, D), lambda i: (0, 0))


_out_rows = jax.ShapeDtypeStruct((N, D), jnp.float32)

_tc_in = pl.pallas_call(
    _tc_in_body, grid=(N // BR,),
    in_specs=[_rows(), _mat(), _vec(), _mat(), _vec()],
    out_specs=_rows(), out_shape=_out_rows)

_tc_mid = pl.pallas_call(
    _tc_mid_body, grid=(N // BR,),
    in_specs=[_rows(), _rows(), _rows(), _vec(), _mat(), _vec()],
    out_specs=_rows(), out_shape=_out_rows)

_tc_out = pl.pallas_call(
    _tc_out_body, grid=(N // BR,),
    in_specs=[_rows(), _rows(), _rows(), _vec(), _mat(), _vec()],
    out_specs=_rows(), out_shape=_out_rows)


@functools.lru_cache(maxsize=None)
def _make_sc_agg():
    mesh = plsc.VectorSubcoreMesh(core_axis_name="c", subcore_axis_name="s")
    return pl.kernel(
        _sc_agg_body,
        mesh=mesh,
        out_type=jax.ShapeDtypeStruct((NC * N, D), jnp.float32),
        scratch_types=[pltpu.VMEM((CHUNK, D), jnp.float32) for _ in range(2)]
        + [pltpu.VMEM((CHUNK,), jnp.int32) for _ in range(4)]
        + [pltpu.VMEM((ZR, D), jnp.float32)]
        + [pltpu.VMEM_SHARED((N, D), jnp.float32)]
        + [pltpu.SemaphoreType.DMA for _ in range(8)],
    )


def _sc_agg_body(hs_hbm, src_hbm, dst_hbm, out_hbm, *rest):
    rows = rest[0:2]
    sbuf = rest[2:4]
    dbuf = rest[4:6]
    zbuf = rest[6]
    acc_sh = rest[7]
    gs = rest[8:10]
    ss = rest[10:12]
    isx = rest[12:14]
    jsx = rest[14:16]

    cid = lax.axis_index("c")
    sid = lax.axis_index("s")
    wg = cid * NS + sid

    def _zrow(r, carry):
        for c in range(D // 16):
            zbuf[r, pl.ds(c * 16, 16)] = jnp.zeros((16,), jnp.float32)
        return carry

    lax.fori_loop(0, ZR, _zrow, 0)

    base = pl.multiple_of(sid * RPW, 16)

    def _zcp(j, carry):
        pltpu.sync_copy(zbuf, acc_sh.at[pl.ds(base + j * ZR, ZR)])
        return carry

    lax.fori_loop(0, RPW // ZR, _zcp, 0)

    @pl.when(sid == NS - 1)
    def _zero_tail():
        pltpu.sync_copy(zbuf, acc_sh.at[pl.ds(NS * RPW, N - NS * RPW)])

    plsc.subcore_barrier()

    def fire_isrc(c, b):
        pltpu.async_copy(src_hbm.at[wg, c], sbuf[b], isx[b])

    def wait_isrc(b):
        pltpu.make_async_copy(src_hbm.at[wg, 0], sbuf[b], isx[b]).wait()

    def fire_idst(c, b):
        pltpu.async_copy(dst_hbm.at[wg, c], dbuf[b], jsx[b])

    def wait_idst(b):
        pltpu.make_async_copy(dst_hbm.at[wg, 0], dbuf[b], jsx[b]).wait()

    def fire_gather(b):
        pltpu.async_copy(hs_hbm.at[sbuf[b]], rows[b], gs[b])

    def wait_gather(b):
        pltpu.make_async_copy(hs_hbm.at[sbuf[b]], rows[b], gs[b]).wait()

    def fire_scatter(b):
        pltpu.async_copy(rows[b], acc_sh.at[dbuf[b]], ss[b], add=True)

    def wait_scatter(b):
        pltpu.make_async_copy(rows[b], acc_sh.at[dbuf[b]], ss[b]).wait()

    for b in range(2):
        fire_isrc(b, b)
        fire_idst(b, b)
    for b in range(2):
        wait_isrc(b)
        fire_gather(b)

    def _pair(k, carry):
        for b in range(2):
            c = 2 * k + b
            wait_gather(b)
            wait_idst(b)
            fire_scatter(b)

            @pl.when(c + 2 < NCHUNK)
            def _pre(b=b, c=c):
                fire_isrc(c + 2, b)

            wait_scatter(b)

            @pl.when(c + 2 < NCHUNK)
            def _nxt(b=b, c=c):
                fire_idst(c + 2, b)
                wait_isrc(b)
                fire_gather(b)
        return carry

    lax.fori_loop(0, (NCHUNK - 1) // 2, _pair, 0)

    wait_gather(0)
    wait_idst(0)
    fire_scatter(0)
    wait_scatter(0)

    plsc.subcore_barrier()
    obase = pl.multiple_of(cid * N + sid * RPW, 16)
    pltpu.sync_copy(acc_sh.at[pl.ds(base, RPW)], out_hbm.at[pl.ds(obase, RPW)])

    @pl.when(sid == NS - 1)
    def _write_tail():
        pltpu.sync_copy(acc_sh.at[pl.ds(NS * RPW, N - NS * RPW)],
                        out_hbm.at[pl.ds(cid * N + NS * RPW, N - NS * RPW)])


def kernel(x, edge_index, W_in, b_in, Ws, bs, eps, W_out, b_out):
    src = edge_index[0].astype(jnp.int32).reshape(NW, NCHUNK, CHUNK)
    dst = edge_index[1].astype(jnp.int32).reshape(NW, NCHUNK, CHUNK)
    ones = jnp.ones((1, D), jnp.float32)

    hs = _tc_in(x, W_in, b_in.reshape(1, D), Ws[0], bs[0].reshape(1, D))
    for i in range(L):
        part = _make_sc_agg()(hs, src, dst)
        p0, p1 = part[:N], part[N:]
        epsb = (1.0 + eps[i]) * ones
        if i < L - 1:
            hs = _tc_mid(hs, p0, p1, epsb, Ws[i + 1], bs[i + 1].reshape(1, D))
        else:
            return _tc_out(hs, p0, p1, epsb, W_out, b_out.reshape(1, D))


# speedup vs baseline: 11.9893x; 1.2890x over previous
import functools

import jax
import jax.numpy as jnp
from jax import lax
from jax.experimental import pallas as pl
from jax.experimental.pallas import tpu as pltpu
from jax.experimental.pallas import tpu_sc as plsc

N = 10000
E = 320000
D = 128
L = 3
K = 32

NC = 2
NS = 16
NW = NC * NS
EPW = E // NW
CHUNK = 80
NCHUNK = EPW // CHUNK
RPW = 624
ZR = 16

BR = 1000


def _maxk_mask(t, _roll=None):
    if _roll is None:
        _roll = lambda v, s: pltpu.roll(v, s, 1)
    n = t.shape[1]
    cols = lax.broadcasted_iota(jnp.int32, t.shape, 1)
    x = t
    k = 2
    while k <= n:
        kbit0 = (cols & k) == 0
        j = k // 2
        while j >= 1:
            jbit0 = (cols & j) == 0
            pv = jnp.where(jbit0, _roll(x, n - j), _roll(x, j))
            take_min = kbit0 == jbit0
            x = jnp.where(take_min, jnp.minimum(x, pv), jnp.maximum(x, pv))
            j //= 2
        k *= 2
    thr = lax.slice(x, (0, n - K), (t.shape[0], n - K + 1))
    gt = t > thr
    cnt_gt = jnp.sum(gt.astype(jnp.int32), axis=1, keepdims=True)
    eq = t == thr
    ec = eq.astype(jnp.int32)
    d = 1
    while d < n:
        ec = ec + jnp.where(cols >= d, _roll(ec, d), 0)
        d *= 2
    keep = jnp.logical_or(gt, jnp.logical_and(eq, ec <= K - cnt_gt))
    return jnp.where(keep, t, 0.0)


def _tc_in_body(x_ref, wi_ref, bi_ref, w0_ref, b0_ref, o_ref):
    h0 = jnp.dot(x_ref[...], wi_ref[...], preferred_element_type=jnp.float32)
    h0 = jnp.maximum(h0 + bi_ref[...], 0.0)
    t = jnp.dot(h0, w0_ref[...], preferred_element_type=jnp.float32) + b0_ref[...]
    o_ref[...] = _maxk_mask(t)


def _tc_mid_body(hs_ref, p0_ref, p1_ref, e_ref, w_ref, b_ref, o_ref):
    h = e_ref[...] * hs_ref[...] + p0_ref[...] + p1_ref[...]
    t = jnp.dot(h, w_ref[...], preferred_element_type=jnp.float32) + b_ref[...]
    o_ref[...] = _maxk_mask(t)


def _tc_out_body(hs_ref, p0_ref, p1_ref, e_ref, w_ref, b_ref, o_ref):
    h = e_ref[...] * hs_ref[...] + p0_ref[...] + p1_ref[...]
    o_ref[...] = jnp.dot(h, w_ref[...], preferred_element_type=jnp.float32) + b_ref[...]


def _rows():
    return pl.BlockSpec((BR, D), lambda i: (i, 0))


def _mat():
    return pl.BlockSpec((D, D), lambda i: (0, 0))


def _vec():
    return pl.BlockSpec((1, D), lambda i: (0, 0))


_out_rows = jax.ShapeDtypeStruct((N, D), jnp.float32)

_tc_in = pl.pallas_call(
    _tc_in_body, grid=(N // BR,),
    in_specs=[_rows(), _mat(), _vec(), _mat(), _vec()],
    out_specs=_rows(), out_shape=_out_rows)

_tc_mid = pl.pallas_call(
    _tc_mid_body, grid=(N // BR,),
    in_specs=[_rows(), _rows(), _rows(), _vec(), _mat(), _vec()],
    out_specs=_rows(), out_shape=_out_rows)

_tc_out = pl.pallas_call(
    _tc_out_body, grid=(N // BR,),
    in_specs=[_rows(), _rows(), _rows(), _vec(), _mat(), _vec()],
    out_specs=_rows(), out_shape=_out_rows)


@functools.lru_cache(maxsize=None)
def _make_sc_agg():
    mesh = plsc.VectorSubcoreMesh(core_axis_name="c", subcore_axis_name="s")
    return pl.kernel(
        _sc_agg_body,
        mesh=mesh,
        out_type=jax.ShapeDtypeStruct((NC * N, D), jnp.float32),
        scratch_types=[pltpu.VMEM((CHUNK, D), jnp.float32) for _ in range(2)]
        + [pltpu.VMEM((CHUNK,), jnp.int32) for _ in range(4)]
        + [pltpu.VMEM((ZR, D), jnp.float32)]
        + [pltpu.VMEM_SHARED((N, D), jnp.float32)]
        + [pltpu.SemaphoreType.DMA for _ in range(8)],
    )


def _sc_agg_body(hs_hbm, src_hbm, dst_hbm, out_hbm, *rest):
    rows = rest[0:2]
    sbuf = rest[2:4]
    dbuf = rest[4:6]
    zbuf = rest[6]
    acc_sh = rest[7]
    gs = rest[8:10]
    ss = rest[10:12]
    isx = rest[12:14]
    jsx = rest[14:16]

    cid = lax.axis_index("c")
    sid = lax.axis_index("s")
    wg = cid * NS + sid

    def _zrow(r, carry):
        for c in range(D // 16):
            zbuf[r, pl.ds(c * 16, 16)] = jnp.zeros((16,), jnp.float32)
        return carry

    lax.fori_loop(0, ZR, _zrow, 0)

    base = pl.multiple_of(sid * RPW, 16)

    def _zcp(j, carry):
        pltpu.sync_copy(zbuf, acc_sh.at[pl.ds(base + j * ZR, ZR)])
        return carry

    lax.fori_loop(0, RPW // ZR, _zcp, 0)

    @pl.when(sid == NS - 1)
    def _zero_tail():
        pltpu.sync_copy(zbuf, acc_sh.at[pl.ds(NS * RPW, N - NS * RPW)])

    plsc.subcore_barrier()

    def fire_isrc(c, b):
        pltpu.async_copy(src_hbm.at[wg, c], sbuf[b], isx[b])

    def wait_isrc(b):
        pltpu.make_async_copy(src_hbm.at[wg, 0], sbuf[b], isx[b]).wait()

    def fire_idst(c, b):
        pltpu.async_copy(dst_hbm.at[wg, c], dbuf[b], jsx[b])

    def wait_idst(b):
        pltpu.make_async_copy(dst_hbm.at[wg, 0], dbuf[b], jsx[b]).wait()

    def fire_gather(b):
        pltpu.async_copy(hs_hbm.at[sbuf[b]], rows[b], gs[b])

    def wait_gather(b):
        pltpu.make_async_copy(hs_hbm.at[sbuf[b]], rows[b], gs[b]).wait()

    def fire_scatter(b):
        pltpu.async_copy(rows[b], acc_sh.at[dbuf[b]], ss[b], add=True)

    def wait_scatter(b):
        pltpu.make_async_copy(rows[b], acc_sh.at[dbuf[b]], ss[b]).wait()

    for b in range(2):
        fire_isrc(b, b)
        fire_idst(b, b)
    for b in range(2):
        wait_isrc(b)
        fire_gather(b)

    def _pair(k, carry):
        for b in range(2):
            c = 2 * k + b
            wait_gather(b)
            wait_idst(b)
            fire_scatter(b)

            @pl.when(c + 2 < NCHUNK)
            def _pre(b=b, c=c):
                fire_isrc(c + 2, b)

            wait_scatter(b)

            @pl.when(c + 2 < NCHUNK)
            def _nxt(b=b, c=c):
                fire_idst(c + 2, b)
                wait_isrc(b)
                fire_gather(b)
        return carry

    lax.fori_loop(0, (NCHUNK - 1) // 2, _pair, 0)

    wait_gather(0)
    wait_idst(0)
    fire_scatter(0)
    wait_scatter(0)

    plsc.subcore_barrier()
    obase = pl.multiple_of(cid * N + sid * RPW, 16)
    pltpu.sync_copy(acc_sh.at[pl.ds(base, RPW)], out_hbm.at[pl.ds(obase, RPW)])

    @pl.when(sid == NS - 1)
    def _write_tail():
        pltpu.sync_copy(acc_sh.at[pl.ds(NS * RPW, N - NS * RPW)],
                        out_hbm.at[pl.ds(cid * N + NS * RPW, N - NS * RPW)])


def kernel(x, edge_index, W_in, b_in, Ws, bs, eps, W_out, b_out):
    src = edge_index[0].astype(jnp.int32).reshape(NW, NCHUNK, CHUNK)
    dst = edge_index[1].astype(jnp.int32).reshape(NW, NCHUNK, CHUNK)
    ones = jnp.ones((1, D), jnp.float32)

    hs = _tc_in(x, W_in, b_in.reshape(1, D), Ws[0], bs[0].reshape(1, D))
    for i in range(L):
        part = _make_sc_agg()(hs, src, dst)
        p0, p1 = part[:N], part[N:]
        epsb = (1.0 + eps[i]) * ones
        if i < L - 1:
            hs = _tc_mid(hs, p0, p1, epsb, Ws[i + 1], bs[i + 1].reshape(1, D))
        else:
            return _tc_out(hs, p0, p1, epsb, W_out, b_out.reshape(1, D))


# speedup vs baseline: 12.6314x; 1.0536x over previous
import functools

import jax
import jax.numpy as jnp
from jax import lax
from jax.experimental import pallas as pl
from jax.experimental.pallas import tpu as pltpu
from jax.experimental.pallas import tpu_sc as plsc

N = 10000
E = 320000
D = 128
L = 3
K = 32

NC = 2
NS = 16
NW = NC * NS
EPW = E // NW
CHUNK = 80
NCHUNK = EPW // CHUNK
RPW = 624
ZR = 16

BR = 1000


def _maxk_mask(t, _roll=None):
    if _roll is None:
        _roll = lambda v, s: pltpu.roll(v, s, 1)
    n = t.shape[1]
    cols = lax.broadcasted_iota(jnp.int32, t.shape, 1)
    u = lax.bitcast_convert_type(t, jnp.int32)
    m = u ^ (lax.shift_right_arithmetic(u, 31) & jnp.int32(0x7FFFFFFF))
    key0 = (m & jnp.int32(-128)) | (127 - cols)
    x = key0
    k = 2
    while k <= n:
        kbit0 = (cols & k) == 0
        j = k // 2
        while j >= 1:
            jbit0 = (cols & j) == 0
            pv = jnp.where(jbit0, _roll(x, n - j), _roll(x, j))
            take_min = kbit0 == jbit0
            x = jnp.where(take_min, jnp.minimum(x, pv), jnp.maximum(x, pv))
            j //= 2
        k *= 2
    thr = lax.slice(x, (0, n - K), (t.shape[0], n - K + 1))
    return jnp.where(key0 >= thr, t, 0.0)


def _tc_in_body(x_ref, wi_ref, bi_ref, w0_ref, b0_ref, o_ref):
    h0 = jnp.dot(x_ref[...], wi_ref[...], preferred_element_type=jnp.float32)
    h0 = jnp.maximum(h0 + bi_ref[...], 0.0)
    t = jnp.dot(h0, w0_ref[...], preferred_element_type=jnp.float32) + b0_ref[...]
    o_ref[...] = _maxk_mask(t)


def _tc_mid_body(hs_ref, p0_ref, p1_ref, e_ref, w_ref, b_ref, o_ref):
    h = e_ref[...] * hs_ref[...] + p0_ref[...] + p1_ref[...]
    t = jnp.dot(h, w_ref[...], preferred_element_type=jnp.float32) + b_ref[...]
    o_ref[...] = _maxk_mask(t)


def _tc_out_body(hs_ref, p0_ref, p1_ref, e_ref, w_ref, b_ref, o_ref):
    h = e_ref[...] * hs_ref[...] + p0_ref[...] + p1_ref[...]
    o_ref[...] = jnp.dot(h, w_ref[...], preferred_element_type=jnp.float32) + b_ref[...]


def _rows():
    return pl.BlockSpec((BR, D), lambda i: (i, 0))


def _mat():
    return pl.BlockSpec((D, D), lambda i: (0, 0))


def _vec():
    return pl.BlockSpec((1, D), lambda i: (0, 0))


_out_rows = jax.ShapeDtypeStruct((N, D), jnp.float32)

_tc_in = pl.pallas_call(
    _tc_in_body, grid=(N // BR,),
    in_specs=[_rows(), _mat(), _vec(), _mat(), _vec()],
    out_specs=_rows(), out_shape=_out_rows)

_tc_mid = pl.pallas_call(
    _tc_mid_body, grid=(N // BR,),
    in_specs=[_rows(), _rows(), _rows(), _vec(), _mat(), _vec()],
    out_specs=_rows(), out_shape=_out_rows)

_tc_out = pl.pallas_call(
    _tc_out_body, grid=(N // BR,),
    in_specs=[_rows(), _rows(), _rows(), _vec(), _mat(), _vec()],
    out_specs=_rows(), out_shape=_out_rows)


@functools.lru_cache(maxsize=None)
def _make_sc_agg():
    mesh = plsc.VectorSubcoreMesh(core_axis_name="c", subcore_axis_name="s")
    return pl.kernel(
        _sc_agg_body,
        mesh=mesh,
        out_type=jax.ShapeDtypeStruct((NC * N, D), jnp.float32),
        scratch_types=[pltpu.VMEM((CHUNK, D), jnp.float32) for _ in range(2)]
        + [pltpu.VMEM((CHUNK,), jnp.int32) for _ in range(4)]
        + [pltpu.VMEM((ZR, D), jnp.float32)]
        + [pltpu.VMEM_SHARED((N, D), jnp.float32)]
        + [pltpu.SemaphoreType.DMA for _ in range(8)],
    )


def _sc_agg_body(hs_hbm, src_hbm, dst_hbm, out_hbm, *rest):
    rows = rest[0:2]
    sbuf = rest[2:4]
    dbuf = rest[4:6]
    zbuf = rest[6]
    acc_sh = rest[7]
    gs = rest[8:10]
    ss = rest[10:12]
    isx = rest[12:14]
    jsx = rest[14:16]

    cid = lax.axis_index("c")
    sid = lax.axis_index("s")
    wg = cid * NS + sid

    def _zrow(r, carry):
        for c in range(D // 16):
            zbuf[r, pl.ds(c * 16, 16)] = jnp.zeros((16,), jnp.float32)
        return carry

    lax.fori_loop(0, ZR, _zrow, 0)

    base = pl.multiple_of(sid * RPW, 16)

    def _zcp(j, carry):
        pltpu.sync_copy(zbuf, acc_sh.at[pl.ds(base + j * ZR, ZR)])
        return carry

    lax.fori_loop(0, RPW // ZR, _zcp, 0)

    @pl.when(sid == NS - 1)
    def _zero_tail():
        pltpu.sync_copy(zbuf, acc_sh.at[pl.ds(NS * RPW, N - NS * RPW)])

    plsc.subcore_barrier()

    def fire_isrc(c, b):
        pltpu.async_copy(src_hbm.at[wg, c], sbuf[b], isx[b])

    def wait_isrc(b):
        pltpu.make_async_copy(src_hbm.at[wg, 0], sbuf[b], isx[b]).wait()

    def fire_idst(c, b):
        pltpu.async_copy(dst_hbm.at[wg, c], dbuf[b], jsx[b])

    def wait_idst(b):
        pltpu.make_async_copy(dst_hbm.at[wg, 0], dbuf[b], jsx[b]).wait()

    def fire_gather(b):
        pltpu.async_copy(hs_hbm.at[sbuf[b]], rows[b], gs[b])

    def wait_gather(b):
        pltpu.make_async_copy(hs_hbm.at[sbuf[b]], rows[b], gs[b]).wait()

    def fire_scatter(b):
        pltpu.async_copy(rows[b], acc_sh.at[dbuf[b]], ss[b], add=True)

    def wait_scatter(b):
        pltpu.make_async_copy(rows[b], acc_sh.at[dbuf[b]], ss[b]).wait()

    for b in range(2):
        fire_isrc(b, b)
        fire_idst(b, b)
    for b in range(2):
        wait_isrc(b)
        fire_gather(b)

    def _pair(k, carry):
        for b in range(2):
            c = 2 * k + b
            wait_gather(b)
            wait_idst(b)
            fire_scatter(b)

            @pl.when(c + 2 < NCHUNK)
            def _pre(b=b, c=c):
                fire_isrc(c + 2, b)

            wait_scatter(b)

            @pl.when(c + 2 < NCHUNK)
            def _nxt(b=b, c=c):
                fire_idst(c + 2, b)
                wait_isrc(b)
                fire_gather(b)
        return carry

    lax.fori_loop(0, (NCHUNK - 1) // 2, _pair, 0)

    wait_gather(0)
    wait_idst(0)
    fire_scatter(0)
    wait_scatter(0)

    plsc.subcore_barrier()
    obase = pl.multiple_of(cid * N + sid * RPW, 16)
    pltpu.sync_copy(acc_sh.at[pl.ds(base, RPW)], out_hbm.at[pl.ds(obase, RPW)])

    @pl.when(sid == NS - 1)
    def _write_tail():
        pltpu.sync_copy(acc_sh.at[pl.ds(NS * RPW, N - NS * RPW)],
                        out_hbm.at[pl.ds(cid * N + NS * RPW, N - NS * RPW)])


def kernel(x, edge_index, W_in, b_in, Ws, bs, eps, W_out, b_out):
    src = edge_index[0].astype(jnp.int32).reshape(NW, NCHUNK, CHUNK)
    dst = edge_index[1].astype(jnp.int32).reshape(NW, NCHUNK, CHUNK)
    ones = jnp.ones((1, D), jnp.float32)

    hs = _tc_in(x, W_in, b_in.reshape(1, D), Ws[0], bs[0].reshape(1, D))
    for i in range(L):
        part = _make_sc_agg()(hs, src, dst)
        p0, p1 = part[:N], part[N:]
        epsb = (1.0 + eps[i]) * ones
        if i < L - 1:
            hs = _tc_mid(hs, p0, p1, epsb, Ws[i + 1], bs[i + 1].reshape(1, D))
        else:
            return _tc_out(hs, p0, p1, epsb, W_out, b_out.reshape(1, D))
